# Initial kernel scaffold; baseline (speedup 1.0000x reference)
import jax
import jax.numpy as jnp
from jax.experimental import pallas as pl


def kernel(edge_index, edge_type, current_time, event_triples, offline_semantic_embs, entity_emb, relation_emb, time_freq, time_phase, basis_0, coeff_0, w_self_0, b_0, basis_1, coeff_1, w_self_1, b_1, sp_w1, sp_b1, sp_g, sp_be, sp_w2, sp_b2, wg, bg):
    raise NotImplementedError("write your pallas kernel here")


# speedup vs baseline: 3.8272x; 3.8272x over previous
import functools

import jax
import jax.numpy as jnp
from jax import lax
from jax.experimental import pallas as pl
from jax.experimental.pallas import tpu as pltpu
from jax.experimental.pallas import tpu_sc as plsc

N = 10000
NREL = 230
H = 128
SEM = 768
NB = 10
E = 320000
B = 1024
NC = 2
NS = 16
NW = NC * NS
CH = 80


def _mesh():
    return plsc.VectorSubcoreMesh(core_axis_name="c", subcore_axis_name="s")


def _sc_gather(table, idx, D, total, ch):
    per_w = total // NW

    @functools.partial(
        pl.kernel,
        out_type=jax.ShapeDtypeStruct((total, D), jnp.float32),
        mesh=_mesh(),
        scratch_types=[
            pltpu.VMEM((ch,), jnp.int32),
            pltpu.VMEM((ch, D), jnp.float32),
            pltpu.SemaphoreType.DMA,
        ],
    )
    def k(table_hbm, idx_hbm, out_hbm, idx_v, buf_v, sem):
        wid = lax.axis_index("c") * NS + lax.axis_index("s")
        base = wid * per_w

        @pl.loop(0, per_w, step=ch)
        def _(i):
            pltpu.sync_copy(idx_hbm.at[pl.ds(base + i, ch)], idx_v)
            pltpu.async_copy(table_hbm.at[idx_v], buf_v, sem).wait()
            pltpu.sync_copy(buf_v, out_hbm.at[pl.ds(base + i, ch)])

    return k(table, idx)


def _sc_scatter_add(rows, dstidx, zeros, total, ch, const_rows=False):
    per_core = total // NC
    per_w = per_core // NS

    @functools.partial(
        pl.kernel,
        out_type=jax.ShapeDtypeStruct((NC, N, H), jnp.float32),
        mesh=_mesh(),
        scratch_types=[
            pltpu.VMEM((ch,), jnp.int32),
            pltpu.VMEM((ch, H), jnp.float32),
            pltpu.VMEM_SHARED((N, H), jnp.float32),
            pltpu.SemaphoreType.DMA,
        ],
    )
    def k(rows_hbm, dst_hbm, zeros_hbm, out_hbm, idx_v, buf_v, acc_sh, sem):
        c = lax.axis_index("c")
        s = lax.axis_index("s")
        base = c * per_core + s * per_w

        @pl.when(s == 0)
        def _():
            pltpu.sync_copy(zeros_hbm, acc_sh)

        if const_rows:
            pltpu.sync_copy(rows_hbm, buf_v)

        plsc.subcore_barrier()

        @pl.loop(0, per_w, step=ch)
        def _(i):
            pltpu.sync_copy(dst_hbm.at[pl.ds(base + i, ch)], idx_v)
            if not const_rows:
                pltpu.sync_copy(rows_hbm.at[pl.ds(base + i, ch)], buf_v)
            pltpu.sync_copy(buf_v, acc_sh.at[idx_v], add=True)

        plsc.subcore_barrier()

        @pl.when(s == 0)
        def _():
            pltpu.sync_copy(acc_sh, out_hbm.at[c])

    return k(rows, dstidx, zeros)


def _tc_x0(ent, ct, freq, phase):
    R = 1000

    def body(e_ref, ct_ref, f_ref, p_ref, o_ref):
        t = jnp.sin(ct_ref[0, 0] * f_ref[...] + p_ref[...])
        o_ref[...] = e_ref[...] + t

    return pl.pallas_call(
        body,
        grid=(N // R,),
        in_specs=[
            pl.BlockSpec((R, H), lambda i: (i, 0)),
            pl.BlockSpec((1, 1), lambda i: (0, 0)),
            pl.BlockSpec((1, H), lambda i: (0, 0)),
            pl.BlockSpec((1, H), lambda i: (0, 0)),
        ],
        out_specs=pl.BlockSpec((R, H), lambda i: (i, 0)),
        out_shape=jax.ShapeDtypeStruct((N, H), jnp.float32),
    )(ent, ct.reshape(1, 1), freq.reshape(1, H), phase.reshape(1, H))


def _dot(a, b):
    return jax.lax.dot_general(a, b, (((1,), (0,)), ((), ())),
                               preferred_element_type=jnp.float32)


def _tc_dense(x, bcat, wself, bias):
    R = 1000

    def body(x_ref, bc_ref, ws_ref, b_ref, xb_ref, sf_ref):
        xv = x_ref[...]
        xb_ref[...] = _dot(xv, bc_ref[...])
        sf_ref[...] = _dot(xv, ws_ref[...]) + b_ref[...]

    return pl.pallas_call(
        body,
        grid=(N // R,),
        in_specs=[
            pl.BlockSpec((R, H), lambda i: (i, 0)),
            pl.BlockSpec((H, NB * H), lambda i: (0, 0)),
            pl.BlockSpec((H, H), lambda i: (0, 0)),
            pl.BlockSpec((1, H), lambda i: (0, 0)),
        ],
        out_specs=(
            pl.BlockSpec((R, NB * H), lambda i: (i, 0)),
            pl.BlockSpec((R, H), lambda i: (i, 0)),
        ),
        out_shape=(
            jax.ShapeDtypeStruct((N, NB * H), jnp.float32),
            jax.ShapeDtypeStruct((N, H), jnp.float32),
        ),
    )(x, bcat, wself, bias.reshape(1, H))


def _tc_combine(rows, w, wsl):
    R = 1000

    def body(r_ref, w_ref, o_ref):
        acc = w_ref[:, wsl:wsl + 1] * r_ref[:, 0:H]
        for b in range(1, NB):
            acc = acc + w_ref[:, wsl + b:wsl + b + 1] * r_ref[:, b * H:(b + 1) * H]
        o_ref[...] = acc

    return pl.pallas_call(
        body,
        grid=(E // R,),
        in_specs=[
            pl.BlockSpec((R, NB * H), lambda i: (i, 0)),
            pl.BlockSpec((R, H), lambda i: (i, 0)),
        ],
        out_specs=pl.BlockSpec((R, H), lambda i: (i, 0)),
        out_shape=jax.ShapeDtypeStruct((E, H), jnp.float32),
    )(rows, w)


def _tc_post(aggs, degw, selfb):
    R = 1000

    def body(a_ref, d_ref, s_ref, o_ref):
        agg = a_ref[0] + a_ref[1]
        deg = d_ref[0, :, 0:1] + d_ref[1, :, 0:1]
        normv = 1.0 / jnp.maximum(deg, 1.0)
        o_ref[...] = jnp.maximum(agg * normv + s_ref[...], 0.0)

    return pl.pallas_call(
        body,
        grid=(N // R,),
        in_specs=[
            pl.BlockSpec((2, R, H), lambda i: (0, i, 0)),
            pl.BlockSpec((2, R, H), lambda i: (0, i, 0)),
            pl.BlockSpec((R, H), lambda i: (i, 0)),
        ],
        out_specs=pl.BlockSpec((R, H), lambda i: (i, 0)),
        out_shape=jax.ShapeDtypeStruct((N, H), jnp.float32),
    )(aggs, degw, selfb)


def _tc_events(sg, rg, dg, sem_embs, w1, b1, g, be, w2, b2, wg1, wg2, bgv):
    def body(sg_ref, rg_ref, dg_ref, se_ref, w1_ref, b1_ref, g_ref, be_ref,
             w2_ref, b2_ref, wg1_ref, wg2_ref, bg_ref,
             st_ref, sm_ref, fu_ref):
        st = sg_ref[...] + rg_ref[...] + dg_ref[...]
        h = _dot(se_ref[...], w1_ref[...]) + b1_ref[...]
        mu = jnp.mean(h, axis=1, keepdims=True)
        var = jnp.mean((h - mu) ** 2, axis=1, keepdims=True)
        h = (h - mu) / jnp.sqrt(var + 1e-5) * g_ref[...] + be_ref[...]
        h = jnp.maximum(h, 0.0)
        smv = _dot(h, w2_ref[...]) + b2_ref[...]
        z = _dot(st, wg1_ref[...]) + _dot(smv, wg2_ref[...]) + bg_ref[...]
        gate = 1.0 / (1.0 + jnp.exp(-z))
        st_ref[...] = st
        sm_ref[...] = smv
        fu_ref[...] = gate * st + (1.0 - gate) * smv

    return pl.pallas_call(
        body,
        out_shape=(
            jax.ShapeDtypeStruct((B, H), jnp.float32),
            jax.ShapeDtypeStruct((B, H), jnp.float32),
            jax.ShapeDtypeStruct((B, H), jnp.float32),
        ),
    )(sg, rg, dg, sem_embs, w1, b1.reshape(1, H), g.reshape(1, H),
      be.reshape(1, H), w2, b2.reshape(1, H), wg1, wg2, bgv.reshape(1, H))


def kernel(edge_index, edge_type, current_time, event_triples, offline_semantic_embs,
           entity_emb, relation_emb, time_freq, time_phase,
           basis_0, coeff_0, w_self_0, b_0, basis_1, coeff_1, w_self_1, b_1,
           sp_w1, sp_b1, sp_g, sp_be, sp_w2, sp_b2, wg, bg):
    src = edge_index[0]
    dst = edge_index[1]
    zeros128 = jnp.zeros((N, H), jnp.float32)
    ones_blk = jnp.ones((CH, H), jnp.float32)
    coeff_cat = jnp.zeros((NREL, H), jnp.float32)
    coeff_cat = coeff_cat.at[:, :NB].set(coeff_0).at[:, 64:64 + NB].set(coeff_1)

    x = _tc_x0(entity_emb, current_time, time_freq, time_phase)
    degw = _sc_scatter_add(ones_blk, dst, zeros128, E, CH, const_rows=True)
    wboth = _sc_gather(coeff_cat, edge_type, H, E, CH)

    for basis, wself, bias, wsl in ((basis_0, w_self_0, b_0, 0),
                                    (basis_1, w_self_1, b_1, 64)):
        bc = jnp.transpose(basis, (1, 0, 2)).reshape(H, NB * H)
        xball, selfb = _tc_dense(x, bc, wself, bias)
        rows = _sc_gather(xball, src, NB * H, E, CH)
        msg = _tc_combine(rows, wboth, wsl)
        aggs = _sc_scatter_add(msg, dst, zeros128, E, CH)
        x = _tc_post(aggs, degw, selfb)

    sg = _sc_gather(x, event_triples[:, 0], H, B, 32)
    rg = _sc_gather(relation_emb, event_triples[:, 1], H, B, 32)
    dg = _sc_gather(x, event_triples[:, 2], H, B, 32)
    ev = _tc_events(sg, rg, dg, offline_semantic_embs, sp_w1, sp_b1, sp_g, sp_be,
                    sp_w2, sp_b2, wg[:H], wg[H:], bg)
    return (x, ev[0], ev[1], ev[2])


# speedup vs baseline: 3.9807x; 1.0401x over previous
import functools

import jax
import jax.numpy as jnp
from jax import lax
from jax.experimental import pallas as pl
from jax.experimental.pallas import tpu as pltpu
from jax.experimental.pallas import tpu_sc as plsc

N = 10000
NREL = 230
H = 128
SEM = 768
NB = 10
E = 320000
B = 1024
NC = 2
NS = 16
NW = NC * NS
CH = 80


def _mesh():
    return plsc.VectorSubcoreMesh(core_axis_name="c", subcore_axis_name="s")


def _sc_gather(table, idx, D, total, ch):
    per_w = total // NW

    pipelined = per_w % (2 * ch) == 0

    @functools.partial(
        pl.kernel,
        out_type=jax.ShapeDtypeStruct((total, D), jnp.float32),
        mesh=_mesh(),
        scratch_types=[
            pltpu.VMEM((ch,), jnp.int32),
            pltpu.VMEM((ch,), jnp.int32),
            pltpu.VMEM((ch, D), jnp.float32),
            pltpu.VMEM((ch, D), jnp.float32),
            pltpu.SemaphoreType.DMA,
            pltpu.SemaphoreType.DMA,
        ],
    )
    def k2(table_hbm, idx_hbm, out_hbm, idx_v0, idx_v1, buf0, buf1, sem0, sem1):
        wid = lax.axis_index("c") * NS + lax.axis_index("s")
        base = wid * per_w

        @pl.loop(0, per_w, step=2 * ch)
        def _(i):
            pltpu.sync_copy(idx_hbm.at[pl.ds(base + i, ch)], idx_v0)
            g0 = pltpu.async_copy(table_hbm.at[idx_v0], buf0, sem0)
            pltpu.sync_copy(idx_hbm.at[pl.ds(base + i + ch, ch)], idx_v1)
            g1 = pltpu.async_copy(table_hbm.at[idx_v1], buf1, sem1)
            g0.wait()
            pltpu.sync_copy(buf0, out_hbm.at[pl.ds(base + i, ch)])
            g1.wait()
            pltpu.sync_copy(buf1, out_hbm.at[pl.ds(base + i + ch, ch)])

    @functools.partial(
        pl.kernel,
        out_type=jax.ShapeDtypeStruct((total, D), jnp.float32),
        mesh=_mesh(),
        scratch_types=[
            pltpu.VMEM((ch,), jnp.int32),
            pltpu.VMEM((ch, D), jnp.float32),
            pltpu.SemaphoreType.DMA,
        ],
    )
    def k(table_hbm, idx_hbm, out_hbm, idx_v, buf_v, sem):
        wid = lax.axis_index("c") * NS + lax.axis_index("s")
        base = wid * per_w

        @pl.loop(0, per_w, step=ch)
        def _(i):
            pltpu.sync_copy(idx_hbm.at[pl.ds(base + i, ch)], idx_v)
            pltpu.async_copy(table_hbm.at[idx_v], buf_v, sem).wait()
            pltpu.sync_copy(buf_v, out_hbm.at[pl.ds(base + i, ch)])

    return (k2 if pipelined else k)(table, idx)


def _sc_scatter_add(rows, dstidx, zeros, total, ch, const_rows=False):
    per_core = total // NC
    per_w = per_core // NS

    @functools.partial(
        pl.kernel,
        out_type=jax.ShapeDtypeStruct((NC, N, H), jnp.float32),
        mesh=_mesh(),
        scratch_types=[
            pltpu.VMEM((ch,), jnp.int32),
            pltpu.VMEM((ch, H), jnp.float32),
            pltpu.VMEM_SHARED((N, H), jnp.float32),
            pltpu.SemaphoreType.DMA,
        ],
    )
    def k(rows_hbm, dst_hbm, zeros_hbm, out_hbm, idx_v, buf_v, acc_sh, sem):
        c = lax.axis_index("c")
        s = lax.axis_index("s")
        base = c * per_core + s * per_w

        @pl.when(s == 0)
        def _():
            pltpu.sync_copy(zeros_hbm, acc_sh)

        if const_rows:
            pltpu.sync_copy(rows_hbm, buf_v)

        plsc.subcore_barrier()

        @pl.loop(0, per_w, step=ch)
        def _(i):
            pltpu.sync_copy(dst_hbm.at[pl.ds(base + i, ch)], idx_v)
            if not const_rows:
                pltpu.sync_copy(rows_hbm.at[pl.ds(base + i, ch)], buf_v)
            pltpu.sync_copy(buf_v, acc_sh.at[idx_v], add=True)

        plsc.subcore_barrier()

        @pl.when(s == 0)
        def _():
            pltpu.sync_copy(acc_sh, out_hbm.at[c])

    return k(rows, dstidx, zeros)


def _tc_x0(ent, ct, freq, phase):
    R = 1000

    def body(e_ref, ct_ref, f_ref, p_ref, o_ref):
        t = jnp.sin(ct_ref[0, 0] * f_ref[...] + p_ref[...])
        o_ref[...] = e_ref[...] + t

    return pl.pallas_call(
        body,
        grid=(N // R,),
        in_specs=[
            pl.BlockSpec((R, H), lambda i: (i, 0)),
            pl.BlockSpec((1, 1), lambda i: (0, 0)),
            pl.BlockSpec((1, H), lambda i: (0, 0)),
            pl.BlockSpec((1, H), lambda i: (0, 0)),
        ],
        out_specs=pl.BlockSpec((R, H), lambda i: (i, 0)),
        out_shape=jax.ShapeDtypeStruct((N, H), jnp.float32),
    )(ent, ct.reshape(1, 1), freq.reshape(1, H), phase.reshape(1, H))


def _dot(a, b):
    return jax.lax.dot_general(a, b, (((1,), (0,)), ((), ())),
                               preferred_element_type=jnp.float32)


def _tc_dense(x, bcat, wself, bias):
    R = 1000

    def body(x_ref, bc_ref, ws_ref, b_ref, xb_ref, sf_ref):
        xv = x_ref[...]
        xb_ref[...] = _dot(xv, bc_ref[...])
        sf_ref[...] = _dot(xv, ws_ref[...]) + b_ref[...]

    return pl.pallas_call(
        body,
        grid=(N // R,),
        in_specs=[
            pl.BlockSpec((R, H), lambda i: (i, 0)),
            pl.BlockSpec((H, NB * H), lambda i: (0, 0)),
            pl.BlockSpec((H, H), lambda i: (0, 0)),
            pl.BlockSpec((1, H), lambda i: (0, 0)),
        ],
        out_specs=(
            pl.BlockSpec((R, NB * H), lambda i: (i, 0)),
            pl.BlockSpec((R, H), lambda i: (i, 0)),
        ),
        out_shape=(
            jax.ShapeDtypeStruct((N, NB * H), jnp.float32),
            jax.ShapeDtypeStruct((N, H), jnp.float32),
        ),
    )(x, bcat, wself, bias.reshape(1, H))


def _tc_combine(rows, w, wsl):
    R = 1000

    def body(r_ref, w_ref, o_ref):
        acc = w_ref[:, wsl:wsl + 1] * r_ref[:, 0:H]
        for b in range(1, NB):
            acc = acc + w_ref[:, wsl + b:wsl + b + 1] * r_ref[:, b * H:(b + 1) * H]
        o_ref[...] = acc

    return pl.pallas_call(
        body,
        grid=(E // R,),
        in_specs=[
            pl.BlockSpec((R, NB * H), lambda i: (i, 0)),
            pl.BlockSpec((R, H), lambda i: (i, 0)),
        ],
        out_specs=pl.BlockSpec((R, H), lambda i: (i, 0)),
        out_shape=jax.ShapeDtypeStruct((E, H), jnp.float32),
    )(rows, w)


def _tc_post(aggs, degw, selfb):
    R = 1000

    def body(a_ref, d_ref, s_ref, o_ref):
        agg = a_ref[0] + a_ref[1]
        deg = d_ref[0, :, 0:1] + d_ref[1, :, 0:1]
        normv = 1.0 / jnp.maximum(deg, 1.0)
        o_ref[...] = jnp.maximum(agg * normv + s_ref[...], 0.0)

    return pl.pallas_call(
        body,
        grid=(N // R,),
        in_specs=[
            pl.BlockSpec((2, R, H), lambda i: (0, i, 0)),
            pl.BlockSpec((2, R, H), lambda i: (0, i, 0)),
            pl.BlockSpec((R, H), lambda i: (i, 0)),
        ],
        out_specs=pl.BlockSpec((R, H), lambda i: (i, 0)),
        out_shape=jax.ShapeDtypeStruct((N, H), jnp.float32),
    )(aggs, degw, selfb)


def _tc_events(sg, rg, dg, sem_embs, w1, b1, g, be, w2, b2, wg1, wg2, bgv):
    def body(sg_ref, rg_ref, dg_ref, se_ref, w1_ref, b1_ref, g_ref, be_ref,
             w2_ref, b2_ref, wg1_ref, wg2_ref, bg_ref,
             st_ref, sm_ref, fu_ref):
        st = sg_ref[...] + rg_ref[...] + dg_ref[...]
        h = _dot(se_ref[...], w1_ref[...]) + b1_ref[...]
        mu = jnp.mean(h, axis=1, keepdims=True)
        var = jnp.mean((h - mu) ** 2, axis=1, keepdims=True)
        h = (h - mu) / jnp.sqrt(var + 1e-5) * g_ref[...] + be_ref[...]
        h = jnp.maximum(h, 0.0)
        smv = _dot(h, w2_ref[...]) + b2_ref[...]
        z = _dot(st, wg1_ref[...]) + _dot(smv, wg2_ref[...]) + bg_ref[...]
        gate = 1.0 / (1.0 + jnp.exp(-z))
        st_ref[...] = st
        sm_ref[...] = smv
        fu_ref[...] = gate * st + (1.0 - gate) * smv

    return pl.pallas_call(
        body,
        out_shape=(
            jax.ShapeDtypeStruct((B, H), jnp.float32),
            jax.ShapeDtypeStruct((B, H), jnp.float32),
            jax.ShapeDtypeStruct((B, H), jnp.float32),
        ),
    )(sg, rg, dg, sem_embs, w1, b1.reshape(1, H), g.reshape(1, H),
      be.reshape(1, H), w2, b2.reshape(1, H), wg1, wg2, bgv.reshape(1, H))


def kernel(edge_index, edge_type, current_time, event_triples, offline_semantic_embs,
           entity_emb, relation_emb, time_freq, time_phase,
           basis_0, coeff_0, w_self_0, b_0, basis_1, coeff_1, w_self_1, b_1,
           sp_w1, sp_b1, sp_g, sp_be, sp_w2, sp_b2, wg, bg):
    src = edge_index[0]
    dst = edge_index[1]
    zeros128 = jnp.zeros((N, H), jnp.float32)
    ones_blk = jnp.ones((CH, H), jnp.float32)
    coeff_cat = jnp.zeros((NREL, H), jnp.float32)
    coeff_cat = coeff_cat.at[:, :NB].set(coeff_0).at[:, 64:64 + NB].set(coeff_1)

    x = _tc_x0(entity_emb, current_time, time_freq, time_phase)
    degw = _sc_scatter_add(ones_blk, dst, zeros128, E, CH, const_rows=True)
    wboth = _sc_gather(coeff_cat, edge_type, H, E, 40)

    for basis, wself, bias, wsl in ((basis_0, w_self_0, b_0, 0),
                                    (basis_1, w_self_1, b_1, 64)):
        bc = jnp.transpose(basis, (1, 0, 2)).reshape(H, NB * H)
        xball, selfb = _tc_dense(x, bc, wself, bias)
        rows = _sc_gather(xball, src, NB * H, E, 40)
        msg = _tc_combine(rows, wboth, wsl)
        aggs = _sc_scatter_add(msg, dst, zeros128, E, CH)
        x = _tc_post(aggs, degw, selfb)

    sg = _sc_gather(x, event_triples[:, 0], H, B, 32)
    rg = _sc_gather(relation_emb, event_triples[:, 1], H, B, 32)
    dg = _sc_gather(x, event_triples[:, 2], H, B, 32)
    ev = _tc_events(sg, rg, dg, offline_semantic_embs, sp_w1, sp_b1, sp_g, sp_be,
                    sp_w2, sp_b2, wg[:H], wg[H:], bg)
    return (x, ev[0], ev[1], ev[2])
